# baseline (speedup 1.0000x reference)
import functools
import jax
import jax.numpy as jnp
from jax import lax
from jax.experimental import pallas as pl
from jax.experimental.pallas import tpu as pltpu
from jax.experimental.pallas import tpu_sc as plsc

NN = 10000
NP = 10240
EE = 320000
DI = 128
HH = 32
CC = 10
GG = 64

NC = 2
NS = 16
NW = NC * NS
CHUNK = 128
ECH = EE // CHUNK
CPW = 80
CPW_LAST = ECH - (NW - 1) * CPW
RPS = NP // NS
NBUF = 4

_SC_MESH = plsc.VectorSubcoreMesh(core_axis_name="c", subcore_axis_name="s")
_SC_PARAMS = pltpu.CompilerParams(use_tc_tiling_on_sc=False)


def _worker_info(c, s):
    wid = s * NC + c
    trips = jnp.where(wid == NW - 1, CPW_LAST, CPW)
    return wid, trips


def _sc_deg_body(ei_hbm, zeros_hbm, deg_hbm, idx_dst, ones_v, deg_sp, sem):
    c = lax.axis_index("c")
    s = lax.axis_index("s")
    wid, trips = _worker_info(c, s)
    pltpu.sync_copy(zeros_hbm.at[pl.ds(s * RPS, RPS)],
                    deg_sp.at[pl.ds(s * RPS, RPS)])

    @pl.when(wid < NW - 1)
    def _full():
        pltpu.async_copy(ei_hbm.at[pl.ds(wid * CPW, CPW)], idx_dst,
                         sem).wait()

    @pl.when(wid == NW - 1)
    def _tail():
        pltpu.async_copy(ei_hbm.at[pl.ds(wid * CPW, CPW_LAST)],
                         idx_dst.at[pl.ds(0, CPW_LAST)], sem).wait()

    for i in range(CHUNK // 16):
        ones_v[pl.ds(i * 16, 16)] = jnp.ones((16,), jnp.float32)
    plsc.subcore_barrier()

    def body(j, carry):
        pltpu.async_copy(ones_v, deg_sp.at[idx_dst.at[j]], sem, add=True)
        return carry

    lax.fori_loop(0, trips, body, 0)

    def drain(j, carry):
        pltpu.make_async_copy(ones_v, deg_sp.at[idx_dst.at[0]], sem).wait()
        return carry

    lax.fori_loop(0, trips, drain, 0)
    plsc.subcore_barrier()
    pltpu.sync_copy(deg_sp.at[pl.ds(s * RPS, RPS)],
                    deg_hbm.at[c, pl.ds(s * RPS, RPS)])


_sc_deg = functools.partial(
    pl.kernel,
    out_type=jax.ShapeDtypeStruct((NC, NP), jnp.float32),
    mesh=_SC_MESH,
    scratch_types=[
        pltpu.VMEM((CPW, CHUNK), jnp.int32),
        pltpu.VMEM((CHUNK,), jnp.float32),
        pltpu.VMEM_SHARED((NP,), jnp.float32),
        pltpu.SemaphoreType.DMA,
    ],
    compiler_params=_SC_PARAMS,
)(_sc_deg_body)


def _sc_edge_body(src_hbm, dst_hbm, g_hbm, zeros_hbm, acc_hbm,
                  idx_src, idx_dst, rows, acc_sp, gsems, ssems):
    c = lax.axis_index("c")
    s = lax.axis_index("s")
    wid, trips = _worker_info(c, s)
    g_nodes = g_hbm
    pltpu.sync_copy(zeros_hbm.at[pl.ds(s * RPS, RPS)],
                    acc_sp.at[pl.ds(s * RPS, RPS)])

    @pl.when(wid < NW - 1)
    def _full():
        pltpu.async_copy(src_hbm.at[pl.ds(wid * CPW, CPW)], idx_src,
                         gsems.at[0]).wait()
        pltpu.async_copy(dst_hbm.at[pl.ds(wid * CPW, CPW)], idx_dst,
                         gsems.at[0]).wait()

    @pl.when(wid == NW - 1)
    def _tail():
        pltpu.async_copy(dst_hbm.at[pl.ds(wid * CPW, CPW_LAST)],
                         idx_dst.at[pl.ds(0, CPW_LAST)], gsems.at[0]).wait()
        pltpu.async_copy(src_hbm.at[pl.ds(wid * CPW, CPW_LAST)],
                         idx_src.at[pl.ds(0, CPW_LAST)], gsems.at[0]).wait()

    plsc.subcore_barrier()
    for b in range(NBUF):
        pltpu.async_copy(g_nodes.at[idx_src.at[b]], rows.at[b], gsems.at[b])

    def body(i, carry):
        for b in range(NBUF):
            j = i * NBUF + b
            pltpu.make_async_copy(g_nodes.at[idx_src.at[0]], rows.at[b],
                                  gsems.at[b]).wait()
            pltpu.async_copy(rows.at[b], acc_sp.at[idx_dst.at[j]],
                             ssems.at[b], add=True)
        for b in range(NBUF):
            j = i * NBUF + b
            pltpu.make_async_copy(rows.at[b], acc_sp.at[pl.ds(0, CHUNK)],
                                  ssems.at[b]).wait()

            @pl.when(j + NBUF < trips)
            def _refill():
                pltpu.async_copy(g_nodes.at[idx_src.at[j + NBUF]], rows.at[b],
                                 gsems.at[b])
        return carry

    lax.fori_loop(0, trips // NBUF, body, 0)
    plsc.subcore_barrier()
    pltpu.sync_copy(acc_sp.at[pl.ds(s * RPS, RPS)],
                    acc_hbm.at[c, pl.ds(s * RPS, RPS)])


_sc_edge = functools.partial(
    pl.kernel,
    out_type=jax.ShapeDtypeStruct((NC, NP, HH), jnp.float32),
    mesh=_SC_MESH,
    scratch_types=[
        pltpu.VMEM((CPW, CHUNK), jnp.int32),
        pltpu.VMEM((CPW, CHUNK), jnp.int32),
        pltpu.VMEM((NBUF, CHUNK, HH), jnp.float32),
        pltpu.VMEM_SHARED((NP, HH), jnp.float32),
        pltpu.SemaphoreType.DMA((NBUF,)),
        pltpu.SemaphoreType.DMA((NBUF,)),
    ],
    compiler_params=_SC_PARAMS,
)(_sc_edge_body)


BLK4 = 256


def _tc_pre_body(deg_ref, x_ref, w_ref, s1_ref, s2_ref, g_ref, dinv_ref):
    dinv = lax.rsqrt(deg_ref[0] + deg_ref[1] + 1.0).reshape(4 * BLK4, 1)
    dinv4 = jnp.dot(s1_ref[...], dinv * s2_ref[...],
                    preferred_element_type=jnp.float32)
    h = jnp.dot(x_ref[...], w_ref[...], preferred_element_type=jnp.float32)
    dinv_ref[...] = dinv4
    g_ref[...] = dinv4 * h


def _tc_pre(deg2, xp4, W4, S1, S2):
    return pl.pallas_call(
        _tc_pre_body,
        grid=(NP // 4 // BLK4,),
        in_specs=[
            pl.BlockSpec((NC, 4 * BLK4), lambda i: (0, i)),
            pl.BlockSpec((BLK4, 4 * DI), lambda i: (i, 0)),
            pl.BlockSpec((4 * DI, CHUNK), lambda i: (0, 0)),
            pl.BlockSpec((BLK4, 4 * BLK4), lambda i: (0, 0)),
            pl.BlockSpec((4 * BLK4, CHUNK), lambda i: (0, 0)),
        ],
        out_specs=[
            pl.BlockSpec((BLK4, CHUNK), lambda i: (i, 0)),
            pl.BlockSpec((BLK4, CHUNK), lambda i: (i, 0)),
        ],
        out_shape=[
            jax.ShapeDtypeStruct((NP // 4, CHUNK), jnp.float32),
            jax.ShapeDtypeStruct((NP // 4, CHUNK), jnp.float32),
        ],
    )(deg2, xp4, W4, S1, S2)


def _norm_relu4(acc_ref, g_ref, dinv_ref, b_ref, od_ref):
    g = g_ref[...]
    pre = dinv_ref[...] * (acc_ref[0] + acc_ref[1] + g) + b_ref[...]
    nrm2 = jnp.dot(pre * pre, od_ref[...], preferred_element_type=jnp.float32)
    nrm = jnp.sqrt(nrm2)
    return jnp.maximum(pre / jnp.maximum(nrm, 1e-12), 0.0)


def _tc_post_body(acc_ref, g_ref, dinv_ref, b_ref, od_ref, w_ref, gnext_ref):
    o = _norm_relu4(acc_ref, g_ref, dinv_ref, b_ref, od_ref)
    gnext_ref[...] = dinv_ref[...] * jnp.dot(
        o, w_ref[...], preferred_element_type=jnp.float32)


def _tc_post(acc4, g4, dinv4, b4, od, W4n):
    return pl.pallas_call(
        _tc_post_body,
        grid=(NP // 4 // BLK4,),
        in_specs=[
            pl.BlockSpec((NC, BLK4, CHUNK), lambda i: (0, i, 0)),
            pl.BlockSpec((BLK4, CHUNK), lambda i: (i, 0)),
            pl.BlockSpec((BLK4, CHUNK), lambda i: (i, 0)),
            pl.BlockSpec((1, CHUNK), lambda i: (0, 0)),
            pl.BlockSpec((CHUNK, CHUNK), lambda i: (0, 0)),
            pl.BlockSpec((CHUNK, CHUNK), lambda i: (0, 0)),
        ],
        out_specs=pl.BlockSpec((BLK4, CHUNK), lambda i: (i, 0)),
        out_shape=jax.ShapeDtypeStruct((NP // 4, CHUNK), jnp.float32),
    )(acc4, g4, dinv4, b4, od, W4n)


def _tc_readout_body(acc_ref, g_ref, dinv_ref, b_ref, od_ref, batch_ref,
                     linw_ref, linb_ref, out_ref, pool_s):
    i = pl.program_id(0)
    nblk = pl.num_programs(0)

    @pl.when(i == 0)
    def _init():
        pool_s[...] = jnp.zeros_like(pool_s)

    o = _norm_relu4(acc_ref, g_ref, dinv_ref, b_ref, od_ref)
    batch = batch_ref[...]
    lo = batch_ref[0, 0]
    hi = batch_ref[BLK4 - 1, CHUNK - 1]

    def fold(v, op):
        return op(op(v[:, 0:HH], v[:, HH:2 * HH]),
                  op(v[:, 2 * HH:3 * HH], v[:, 3 * HH:4 * HH]))

    def body(gid, carry):
        m = batch == gid
        vals = jnp.where(m, o, 0.0)
        mx = fold(jnp.max(vals, axis=0, keepdims=True), jnp.maximum)
        sm = fold(jnp.sum(vals, axis=0, keepdims=True), jnp.add)
        ct = fold(jnp.sum(jnp.where(m, 1.0, 0.0), axis=0, keepdims=True),
                  jnp.add)
        old = pool_s[pl.ds(gid, 1), :]
        upd = jnp.concatenate(
            [jnp.maximum(old[:, 0:HH], mx), old[:, HH:2 * HH] + sm,
             old[:, 2 * HH:3 * HH] + ct, old[:, 3 * HH:]], axis=1)
        pool_s[pl.ds(gid, 1), :] = upd
        return carry

    lax.fori_loop(lo, hi + 1, body, 0)

    @pl.when(i == nblk - 1)
    def _fin():
        mean = pool_s[0:GG, HH:2 * HH] / \
            jnp.maximum(pool_s[0:GG, 2 * HH:3 * HH], 1.0)
        pooled = jnp.concatenate([pool_s[0:GG, 0:HH], mean], axis=1)
        out_ref[...] = jnp.dot(
            pooled, linw_ref[...], preferred_element_type=jnp.float32
        ) + linb_ref[...]


def _tc_readout(acc4, g4, dinv4, b4, od, batch4, linW, linb):
    return pl.pallas_call(
        _tc_readout_body,
        grid=(NP // 4 // BLK4,),
        in_specs=[
            pl.BlockSpec((NC, BLK4, CHUNK), lambda i: (0, i, 0)),
            pl.BlockSpec((BLK4, CHUNK), lambda i: (i, 0)),
            pl.BlockSpec((BLK4, CHUNK), lambda i: (i, 0)),
            pl.BlockSpec((1, CHUNK), lambda i: (0, 0)),
            pl.BlockSpec((CHUNK, CHUNK), lambda i: (0, 0)),
            pl.BlockSpec((BLK4, CHUNK), lambda i: (i, 0)),
            pl.BlockSpec((2 * HH, CC), lambda i: (0, 0)),
            pl.BlockSpec((1, CC), lambda i: (0, 0)),
        ],
        out_specs=pl.BlockSpec((GG, CC), lambda i: (0, 0)),
        out_shape=jax.ShapeDtypeStruct((GG, CC), jnp.float32),
        scratch_shapes=[
            pltpu.VMEM((GG + 8, CHUNK), jnp.float32),
        ],
    )(acc4, g4, dinv4, b4, od, batch4, linW, linb)


def kernel(x, edge_index, batch, W1, b1, W2, b2, W3, b3, linW, linb):
    f32 = jnp.float32
    xp4 = jnp.zeros((NP, DI), f32).at[:NN].set(x).reshape(NP // 4, 4 * DI)
    batch4 = jnp.broadcast_to(
        jnp.full((NP,), GG, jnp.int32).at[:NN].set(batch)[:, None],
        (NP, HH)).reshape(NP // 4, CHUNK)
    zeros_nh = jnp.zeros((NP, HH), f32)
    zeros_n = jnp.zeros((NP,), f32)
    src2d = edge_index[0].reshape(ECH, CHUNK)
    dst2d = edge_index[1].reshape(ECH, CHUNK)

    eye4 = jnp.eye(4, dtype=f32)
    od = jnp.kron(eye4, jnp.ones((HH, HH), f32))
    W4_1 = jnp.kron(eye4, W1)
    W4_2 = jnp.kron(eye4, W2)
    W4_3 = jnp.kron(eye4, W3)
    S1 = jnp.kron(jnp.eye(BLK4, dtype=f32), jnp.ones((1, 4), f32))
    S2 = jnp.kron(jnp.ones((BLK4, 1), f32),
                  jnp.kron(eye4, jnp.ones((1, HH), f32)))

    def b4(b):
        return jnp.tile(b, 4).reshape(1, CHUNK)

    def wide(a):
        return a.reshape(a.shape[:-2] + (NP // 4, CHUNK))

    deg2 = _sc_deg(dst2d, zeros_n)
    g1w, dinv4 = _tc_pre(deg2, xp4, W4_1, S1, S2)
    acc1 = _sc_edge(src2d, dst2d, g1w.reshape(NP, HH), zeros_nh)
    g2w = _tc_post(wide(acc1), g1w, dinv4, b4(b1), od, W4_2)
    acc2 = _sc_edge(src2d, dst2d, g2w.reshape(NP, HH), zeros_nh)
    g3w = _tc_post(wide(acc2), g2w, dinv4, b4(b2), od, W4_3)
    acc3 = _sc_edge(src2d, dst2d, g3w.reshape(NP, HH), zeros_nh)
    return _tc_readout(wide(acc3), g3w, dinv4, b4(b3), od, batch4,
                       linW, linb.reshape(1, CC))


# speedup vs baseline: 1.0689x; 1.0689x over previous
import functools
import jax
import jax.numpy as jnp
from jax import lax
from jax.experimental import pallas as pl
from jax.experimental.pallas import tpu as pltpu
from jax.experimental.pallas import tpu_sc as plsc

NN = 10000
NP = 10240
EE = 320000
DI = 128
HH = 32
CC = 10
GG = 64

NC = 2
NS = 16
NW = NC * NS
CHUNK = 128
ECH = EE // CHUNK
CPW = 80
CPW_LAST = ECH - (NW - 1) * CPW
RPS = NP // NS
NBUF = 4

_SC_MESH = plsc.VectorSubcoreMesh(core_axis_name="c", subcore_axis_name="s")
_SC_PARAMS = pltpu.CompilerParams(use_tc_tiling_on_sc=False)


def _worker_info(c, s):
    wid = s * NC + c
    trips = jnp.where(wid == NW - 1, CPW_LAST, CPW)
    return wid, trips


def _sc_deg_body(ei_hbm, zeros_hbm, deg_hbm, idx_dst, ones_v, deg_sp, sem):
    c = lax.axis_index("c")
    s = lax.axis_index("s")
    wid, trips = _worker_info(c, s)
    pltpu.sync_copy(zeros_hbm.at[pl.ds(s * RPS, RPS)],
                    deg_sp.at[pl.ds(s * RPS, RPS)])

    @pl.when(wid < NW - 1)
    def _full():
        pltpu.async_copy(ei_hbm.at[pl.ds(ECH + wid * CPW, CPW)], idx_dst,
                         sem).wait()

    @pl.when(wid == NW - 1)
    def _tail():
        pltpu.async_copy(ei_hbm.at[pl.ds(ECH + wid * CPW, CPW_LAST)],
                         idx_dst.at[pl.ds(0, CPW_LAST)], sem).wait()

    for i in range(CHUNK // 16):
        ones_v[pl.ds(i * 16, 16)] = jnp.ones((16,), jnp.float32)
    plsc.subcore_barrier()

    def body(j, carry):
        pltpu.async_copy(ones_v, deg_sp.at[idx_dst.at[j]], sem, add=True)
        return carry

    lax.fori_loop(0, trips, body, 0)

    def drain(j, carry):
        pltpu.make_async_copy(ones_v, deg_sp.at[idx_dst.at[0]], sem).wait()
        return carry

    lax.fori_loop(0, trips, drain, 0)
    plsc.subcore_barrier()
    pltpu.sync_copy(deg_sp.at[pl.ds(s * RPS, RPS)],
                    deg_hbm.at[c, pl.ds(s * RPS, RPS)])


_sc_deg = functools.partial(
    pl.kernel,
    out_type=jax.ShapeDtypeStruct((NC, NP), jnp.float32),
    mesh=_SC_MESH,
    scratch_types=[
        pltpu.VMEM((CPW, CHUNK), jnp.int32),
        pltpu.VMEM((CHUNK,), jnp.float32),
        pltpu.VMEM_SHARED((NP,), jnp.float32),
        pltpu.SemaphoreType.DMA,
    ],
    compiler_params=_SC_PARAMS,
)(_sc_deg_body)


def _sc_edge_body(ei_hbm, g_hbm, zeros_hbm, acc_hbm,
                  idx_src, idx_dst, rows, acc_sp, gsems, ssems):
    c = lax.axis_index("c")
    s = lax.axis_index("s")
    wid, trips = _worker_info(c, s)
    g_nodes = g_hbm
    pltpu.sync_copy(zeros_hbm.at[pl.ds(s * RPS, RPS)],
                    acc_sp.at[pl.ds(s * RPS, RPS)])

    @pl.when(wid < NW - 1)
    def _full():
        pltpu.async_copy(ei_hbm.at[pl.ds(wid * CPW, CPW)], idx_src,
                         gsems.at[0]).wait()
        pltpu.async_copy(ei_hbm.at[pl.ds(ECH + wid * CPW, CPW)], idx_dst,
                         gsems.at[0]).wait()

    @pl.when(wid == NW - 1)
    def _tail():
        pltpu.async_copy(ei_hbm.at[pl.ds(ECH + wid * CPW, CPW_LAST)],
                         idx_dst.at[pl.ds(0, CPW_LAST)], gsems.at[0]).wait()
        pltpu.async_copy(ei_hbm.at[pl.ds(wid * CPW, CPW_LAST)],
                         idx_src.at[pl.ds(0, CPW_LAST)], gsems.at[0]).wait()

    plsc.subcore_barrier()
    for b in range(NBUF):
        pltpu.async_copy(g_nodes.at[idx_src.at[b]], rows.at[b], gsems.at[b])

    def body(i, carry):
        for b in range(NBUF):
            j = i * NBUF + b
            pltpu.make_async_copy(g_nodes.at[idx_src.at[0]], rows.at[b],
                                  gsems.at[b]).wait()
            pltpu.async_copy(rows.at[b], acc_sp.at[idx_dst.at[j]],
                             ssems.at[b], add=True)
        for b in range(NBUF):
            j = i * NBUF + b
            pltpu.make_async_copy(rows.at[b], acc_sp.at[pl.ds(0, CHUNK)],
                                  ssems.at[b]).wait()

            @pl.when(j + NBUF < trips)
            def _refill():
                pltpu.async_copy(g_nodes.at[idx_src.at[j + NBUF]], rows.at[b],
                                 gsems.at[b])
        return carry

    lax.fori_loop(0, trips // NBUF, body, 0)
    plsc.subcore_barrier()
    pltpu.sync_copy(acc_sp.at[pl.ds(s * RPS, RPS)],
                    acc_hbm.at[c, pl.ds(s * RPS, RPS)])


_sc_edge = functools.partial(
    pl.kernel,
    out_type=jax.ShapeDtypeStruct((NC, NP, HH), jnp.float32),
    mesh=_SC_MESH,
    scratch_types=[
        pltpu.VMEM((CPW, CHUNK), jnp.int32),
        pltpu.VMEM((CPW, CHUNK), jnp.int32),
        pltpu.VMEM((NBUF, CHUNK, HH), jnp.float32),
        pltpu.VMEM_SHARED((NP, HH), jnp.float32),
        pltpu.SemaphoreType.DMA((NBUF,)),
        pltpu.SemaphoreType.DMA((NBUF,)),
    ],
    compiler_params=_SC_PARAMS,
)(_sc_edge_body)


BLK4 = 256


def _tc_pre_body(deg_ref, x_ref, w_ref, s1_ref, s2_ref, g_ref, dinv_ref):
    dinv = lax.rsqrt(deg_ref[0] + deg_ref[1] + 1.0).reshape(4 * BLK4, 1)
    dinv4 = jnp.dot(s1_ref[...], dinv * s2_ref[...],
                    preferred_element_type=jnp.float32)
    h = jnp.dot(x_ref[...], w_ref[...], preferred_element_type=jnp.float32)
    dinv_ref[...] = dinv4
    g_ref[...] = dinv4 * h


def _tc_pre(deg2, xp4, W4, S1, S2):
    return pl.pallas_call(
        _tc_pre_body,
        grid=(NP // 4 // BLK4,),
        in_specs=[
            pl.BlockSpec((NC, 4 * BLK4), lambda i: (0, i)),
            pl.BlockSpec((BLK4, 4 * DI), lambda i: (i, 0)),
            pl.BlockSpec((4 * DI, CHUNK), lambda i: (0, 0)),
            pl.BlockSpec((BLK4, 4 * BLK4), lambda i: (0, 0)),
            pl.BlockSpec((4 * BLK4, CHUNK), lambda i: (0, 0)),
        ],
        out_specs=[
            pl.BlockSpec((BLK4, CHUNK), lambda i: (i, 0)),
            pl.BlockSpec((BLK4, CHUNK), lambda i: (i, 0)),
        ],
        out_shape=[
            jax.ShapeDtypeStruct((NP // 4, CHUNK), jnp.float32),
            jax.ShapeDtypeStruct((NP // 4, CHUNK), jnp.float32),
        ],
    )(deg2, xp4, W4, S1, S2)


def _norm_relu4(acc_ref, g_ref, dinv_ref, b_ref, od_ref):
    g = g_ref[...]
    pre = dinv_ref[...] * (acc_ref[0] + acc_ref[1] + g) + b_ref[...]
    nrm2 = jnp.dot(pre * pre, od_ref[...], preferred_element_type=jnp.float32)
    nrm = jnp.sqrt(nrm2)
    return jnp.maximum(pre / jnp.maximum(nrm, 1e-12), 0.0)


def _tc_post_body(acc_ref, g_ref, dinv_ref, b_ref, od_ref, w_ref, gnext_ref):
    o = _norm_relu4(acc_ref, g_ref, dinv_ref, b_ref, od_ref)
    gnext_ref[...] = dinv_ref[...] * jnp.dot(
        o, w_ref[...], preferred_element_type=jnp.float32)


def _tc_post(acc4, g4, dinv4, b4, od, W4n):
    return pl.pallas_call(
        _tc_post_body,
        grid=(NP // 4 // BLK4,),
        in_specs=[
            pl.BlockSpec((NC, BLK4, CHUNK), lambda i: (0, i, 0)),
            pl.BlockSpec((BLK4, CHUNK), lambda i: (i, 0)),
            pl.BlockSpec((BLK4, CHUNK), lambda i: (i, 0)),
            pl.BlockSpec((1, CHUNK), lambda i: (0, 0)),
            pl.BlockSpec((CHUNK, CHUNK), lambda i: (0, 0)),
            pl.BlockSpec((CHUNK, CHUNK), lambda i: (0, 0)),
        ],
        out_specs=pl.BlockSpec((BLK4, CHUNK), lambda i: (i, 0)),
        out_shape=jax.ShapeDtypeStruct((NP // 4, CHUNK), jnp.float32),
    )(acc4, g4, dinv4, b4, od, W4n)


def _tc_readout_body(acc_ref, g_ref, dinv_ref, b_ref, od_ref, batch_ref,
                     linw_ref, linb_ref, out_ref, mx_s, sm_s, ct_s):
    i = pl.program_id(0)
    nblk = pl.num_programs(0)

    @pl.when(i == 0)
    def _init():
        mx_s[...] = jnp.zeros_like(mx_s)
        sm_s[...] = jnp.zeros_like(sm_s)
        ct_s[...] = jnp.zeros_like(ct_s)

    o = _norm_relu4(acc_ref, g_ref, dinv_ref, b_ref, od_ref)
    batch = batch_ref[...]

    iota = lax.broadcasted_iota(jnp.int32, (BLK4, GG), 1)
    ones32 = jnp.ones((BLK4, HH), jnp.float32)
    dims = (((0,), (0,)), ((), ()))
    sm_c = jnp.zeros((GG, HH), jnp.float32)
    ct_c = jnp.zeros((GG, HH), jnp.float32)
    for k in range(4):
        mk = (batch[:, HH * k:HH * k + 1] == iota).astype(jnp.float32)
        ok = o[:, HH * k:HH * (k + 1)]
        sm_c = sm_c + lax.dot_general(
            mk, ok, dims, preferred_element_type=jnp.float32)
        ct_c = ct_c + lax.dot_general(
            mk, ones32, dims, preferred_element_type=jnp.float32)
    sm_s[...] = sm_s[...] + sm_c
    ct_s[...] = ct_s[...] + ct_c

    lo = batch_ref[0, 0]
    hi = batch_ref[BLK4 - 1, CHUNK - 1]

    def fold(v, op):
        return op(op(v[:, 0:HH], v[:, HH:2 * HH]),
                  op(v[:, 2 * HH:3 * HH], v[:, 3 * HH:4 * HH]))

    def body(gid, carry):
        vals = jnp.where(batch == gid, o, 0.0)
        mx = fold(jnp.max(vals, axis=0, keepdims=True), jnp.maximum)
        mx_s[pl.ds(gid, 1), :] = jnp.maximum(mx_s[pl.ds(gid, 1), :], mx)
        return carry

    lax.fori_loop(lo, hi + 1, body, 0)

    @pl.when(i == nblk - 1)
    def _fin():
        mean = sm_s[...] / jnp.maximum(ct_s[...], 1.0)
        pooled = jnp.concatenate([mx_s[0:GG, :], mean], axis=1)
        out_ref[...] = jnp.dot(
            pooled, linw_ref[...], preferred_element_type=jnp.float32
        ) + linb_ref[...]


def _tc_readout(acc4, g4, dinv4, b4, od, batch4, linW, linb):
    return pl.pallas_call(
        _tc_readout_body,
        grid=(NP // 4 // BLK4,),
        in_specs=[
            pl.BlockSpec((NC, BLK4, CHUNK), lambda i: (0, i, 0)),
            pl.BlockSpec((BLK4, CHUNK), lambda i: (i, 0)),
            pl.BlockSpec((BLK4, CHUNK), lambda i: (i, 0)),
            pl.BlockSpec((1, CHUNK), lambda i: (0, 0)),
            pl.BlockSpec((CHUNK, CHUNK), lambda i: (0, 0)),
            pl.BlockSpec((BLK4, CHUNK), lambda i: (i, 0)),
            pl.BlockSpec((2 * HH, CC), lambda i: (0, 0)),
            pl.BlockSpec((1, CC), lambda i: (0, 0)),
        ],
        out_specs=pl.BlockSpec((GG, CC), lambda i: (0, 0)),
        out_shape=jax.ShapeDtypeStruct((GG, CC), jnp.float32),
        scratch_shapes=[
            pltpu.VMEM((GG + 8, HH), jnp.float32),
            pltpu.VMEM((GG, HH), jnp.float32),
            pltpu.VMEM((GG, HH), jnp.float32),
        ],
    )(acc4, g4, dinv4, b4, od, batch4, linW, linb)


def kernel(x, edge_index, batch, W1, b1, W2, b2, W3, b3, linW, linb):
    f32 = jnp.float32
    xp4 = jnp.zeros((NP, DI), f32).at[:NN].set(x).reshape(NP // 4, 4 * DI)
    batch4 = jnp.broadcast_to(
        jnp.full((NP,), GG, jnp.int32).at[:NN].set(batch)[:, None],
        (NP, HH)).reshape(NP // 4, CHUNK)
    zeros_nh = jnp.zeros((NP, HH), f32)
    zeros_n = jnp.zeros((NP,), f32)
    ei2d = edge_index.reshape(2 * ECH, CHUNK)

    eye4 = jnp.eye(4, dtype=f32)
    od = jnp.kron(eye4, jnp.ones((HH, HH), f32))
    W4_1 = jnp.kron(eye4, W1)
    W4_2 = jnp.kron(eye4, W2)
    W4_3 = jnp.kron(eye4, W3)
    S1 = jnp.kron(jnp.eye(BLK4, dtype=f32), jnp.ones((1, 4), f32))
    S2 = jnp.kron(jnp.ones((BLK4, 1), f32),
                  jnp.kron(eye4, jnp.ones((1, HH), f32)))

    def b4(b):
        return jnp.tile(b, 4).reshape(1, CHUNK)

    def wide(a):
        return a.reshape(a.shape[:-2] + (NP // 4, CHUNK))

    deg2 = _sc_deg(ei2d, zeros_n)
    g1w, dinv4 = _tc_pre(deg2, xp4, W4_1, S1, S2)
    acc1 = _sc_edge(ei2d, g1w.reshape(NP, HH), zeros_nh)
    g2w = _tc_post(wide(acc1), g1w, dinv4, b4(b1), od, W4_2)
    acc2 = _sc_edge(ei2d, g2w.reshape(NP, HH), zeros_nh)
    g3w = _tc_post(wide(acc2), g2w, dinv4, b4(b2), od, W4_3)
    acc3 = _sc_edge(ei2d, g3w.reshape(NP, HH), zeros_nh)
    return _tc_readout(wide(acc3), g3w, dinv4, b4(b3), od, batch4,
                       linW, linb.reshape(1, CC))


# speedup vs baseline: 1.1059x; 1.0346x over previous
import functools
import jax
import jax.numpy as jnp
from jax import lax
from jax.experimental import pallas as pl
from jax.experimental.pallas import tpu as pltpu
from jax.experimental.pallas import tpu_sc as plsc

NN = 10000
NP = 10240
EE = 320000
DI = 128
HH = 32
CC = 10
GG = 64

NC = 2
NS = 16
NW = NC * NS
CHUNK = 128
ECH = EE // CHUNK
CPW = 80
CPW_LAST = ECH - (NW - 1) * CPW
RPS = NP // NS
NBUF = 5

_SC_MESH = plsc.VectorSubcoreMesh(core_axis_name="c", subcore_axis_name="s")
_SC_PARAMS = pltpu.CompilerParams(use_tc_tiling_on_sc=False)


def _worker_info(c, s):
    wid = s * NC + c
    trips = jnp.where(wid == NW - 1, CPW_LAST, CPW)
    return wid, trips


def _sc_deg_body(ei_hbm, zeros_hbm, deg_hbm, idx_dst, ones_v, deg_sp, sem):
    c = lax.axis_index("c")
    s = lax.axis_index("s")
    wid, trips = _worker_info(c, s)
    pltpu.sync_copy(zeros_hbm.at[pl.ds(s * RPS, RPS)],
                    deg_sp.at[pl.ds(s * RPS, RPS)])

    @pl.when(wid < NW - 1)
    def _full():
        pltpu.async_copy(ei_hbm.at[pl.ds(ECH + wid * CPW, CPW)], idx_dst,
                         sem).wait()

    @pl.when(wid == NW - 1)
    def _tail():
        pltpu.async_copy(ei_hbm.at[pl.ds(ECH + wid * CPW, CPW_LAST)],
                         idx_dst.at[pl.ds(0, CPW_LAST)], sem).wait()

    for i in range(CHUNK // 16):
        ones_v[pl.ds(i * 16, 16)] = jnp.ones((16,), jnp.float32)
    plsc.subcore_barrier()

    def body(j, carry):
        pltpu.async_copy(ones_v, deg_sp.at[idx_dst.at[j]], sem, add=True)
        return carry

    lax.fori_loop(0, trips, body, 0)

    def drain(j, carry):
        pltpu.make_async_copy(ones_v, deg_sp.at[idx_dst.at[0]], sem).wait()
        return carry

    lax.fori_loop(0, trips, drain, 0)
    plsc.subcore_barrier()
    pltpu.sync_copy(deg_sp.at[pl.ds(s * RPS, RPS)],
                    deg_hbm.at[c, pl.ds(s * RPS, RPS)])


_sc_deg = functools.partial(
    pl.kernel,
    out_type=jax.ShapeDtypeStruct((NC, NP), jnp.float32),
    mesh=_SC_MESH,
    scratch_types=[
        pltpu.VMEM((CPW, CHUNK), jnp.int32),
        pltpu.VMEM((CHUNK,), jnp.float32),
        pltpu.VMEM_SHARED((NP,), jnp.float32),
        pltpu.SemaphoreType.DMA,
    ],
    compiler_params=_SC_PARAMS,
)(_sc_deg_body)


def _sc_edge_body(ei_hbm, g_hbm, zeros_hbm, acc_hbm,
                  idx_src, idx_dst, rows, acc_sp, gsems, ssems):
    c = lax.axis_index("c")
    s = lax.axis_index("s")
    wid, trips = _worker_info(c, s)
    g_nodes = g_hbm
    pltpu.sync_copy(zeros_hbm.at[pl.ds(s * RPS, RPS)],
                    acc_sp.at[pl.ds(s * RPS, RPS)])

    @pl.when(wid < NW - 1)
    def _full():
        pltpu.async_copy(ei_hbm.at[pl.ds(wid * CPW, CPW)], idx_src,
                         gsems.at[0]).wait()
        pltpu.async_copy(ei_hbm.at[pl.ds(ECH + wid * CPW, CPW)], idx_dst,
                         gsems.at[0]).wait()

    @pl.when(wid == NW - 1)
    def _tail():
        pltpu.async_copy(ei_hbm.at[pl.ds(ECH + wid * CPW, CPW_LAST)],
                         idx_dst.at[pl.ds(0, CPW_LAST)], gsems.at[0]).wait()
        pltpu.async_copy(ei_hbm.at[pl.ds(wid * CPW, CPW_LAST)],
                         idx_src.at[pl.ds(0, CPW_LAST)], gsems.at[0]).wait()

    plsc.subcore_barrier()
    for b in range(NBUF):
        pltpu.async_copy(g_nodes.at[idx_src.at[b]], rows.at[b], gsems.at[b])

    def body(i, carry):
        for b in range(NBUF):
            j = i * NBUF + b
            pltpu.make_async_copy(g_nodes.at[idx_src.at[0]], rows.at[b],
                                  gsems.at[b]).wait()
            pltpu.async_copy(rows.at[b], acc_sp.at[idx_dst.at[j]],
                             ssems.at[b], add=True)
        for b in range(NBUF):
            j = i * NBUF + b
            pltpu.make_async_copy(rows.at[b], acc_sp.at[pl.ds(0, CHUNK)],
                                  ssems.at[b]).wait()

            @pl.when(j + NBUF < trips)
            def _refill():
                pltpu.async_copy(g_nodes.at[idx_src.at[j + NBUF]], rows.at[b],
                                 gsems.at[b])
        return carry

    lax.fori_loop(0, trips // NBUF, body, 0)
    plsc.subcore_barrier()
    pltpu.sync_copy(acc_sp.at[pl.ds(s * RPS, RPS)],
                    acc_hbm.at[c, pl.ds(s * RPS, RPS)])


_sc_edge = functools.partial(
    pl.kernel,
    out_type=jax.ShapeDtypeStruct((NC, NP, HH), jnp.float32),
    mesh=_SC_MESH,
    scratch_types=[
        pltpu.VMEM((CPW, CHUNK), jnp.int32),
        pltpu.VMEM((CPW, CHUNK), jnp.int32),
        pltpu.VMEM((NBUF, CHUNK, HH), jnp.float32),
        pltpu.VMEM_SHARED((NP, HH), jnp.float32),
        pltpu.SemaphoreType.DMA((NBUF,)),
        pltpu.SemaphoreType.DMA((NBUF,)),
    ],
    compiler_params=_SC_PARAMS,
)(_sc_edge_body)


BLK4 = 256


def _tc_pre_body(deg_ref, x_ref, w_ref, s1_ref, s2_ref, g_ref, dinv_ref):
    dinv = lax.rsqrt(deg_ref[0] + deg_ref[1] + 1.0).reshape(4 * BLK4, 1)
    dinv4 = jnp.dot(s1_ref[...], dinv * s2_ref[...],
                    preferred_element_type=jnp.float32)
    h = jnp.dot(x_ref[...], w_ref[...], preferred_element_type=jnp.float32)
    dinv_ref[...] = dinv4
    g_ref[...] = dinv4 * h


def _tc_pre(deg2, xp4, W4, S1, S2):
    return pl.pallas_call(
        _tc_pre_body,
        grid=(NP // 4 // BLK4,),
        in_specs=[
            pl.BlockSpec((NC, 4 * BLK4), lambda i: (0, i)),
            pl.BlockSpec((BLK4, 4 * DI), lambda i: (i, 0)),
            pl.BlockSpec((4 * DI, CHUNK), lambda i: (0, 0)),
            pl.BlockSpec((BLK4, 4 * BLK4), lambda i: (0, 0)),
            pl.BlockSpec((4 * BLK4, CHUNK), lambda i: (0, 0)),
        ],
        out_specs=[
            pl.BlockSpec((BLK4, CHUNK), lambda i: (i, 0)),
            pl.BlockSpec((BLK4, CHUNK), lambda i: (i, 0)),
        ],
        out_shape=[
            jax.ShapeDtypeStruct((NP // 4, CHUNK), jnp.float32),
            jax.ShapeDtypeStruct((NP // 4, CHUNK), jnp.float32),
        ],
    )(deg2, xp4, W4, S1, S2)


def _norm_relu4(acc_ref, g_ref, dinv_ref, b_ref, od_ref):
    g = g_ref[...]
    pre = dinv_ref[...] * (acc_ref[0] + acc_ref[1] + g) + b_ref[...]
    nrm2 = jnp.dot(pre * pre, od_ref[...], preferred_element_type=jnp.float32)
    nrm = jnp.sqrt(nrm2)
    return jnp.maximum(pre / jnp.maximum(nrm, 1e-12), 0.0)


def _tc_post_body(acc_ref, g_ref, dinv_ref, b_ref, od_ref, w_ref, gnext_ref):
    o = _norm_relu4(acc_ref, g_ref, dinv_ref, b_ref, od_ref)
    gnext_ref[...] = dinv_ref[...] * jnp.dot(
        o, w_ref[...], preferred_element_type=jnp.float32)


def _tc_post(acc4, g4, dinv4, b4, od, W4n):
    return pl.pallas_call(
        _tc_post_body,
        grid=(NP // 4 // BLK4,),
        in_specs=[
            pl.BlockSpec((NC, BLK4, CHUNK), lambda i: (0, i, 0)),
            pl.BlockSpec((BLK4, CHUNK), lambda i: (i, 0)),
            pl.BlockSpec((BLK4, CHUNK), lambda i: (i, 0)),
            pl.BlockSpec((1, CHUNK), lambda i: (0, 0)),
            pl.BlockSpec((CHUNK, CHUNK), lambda i: (0, 0)),
            pl.BlockSpec((CHUNK, CHUNK), lambda i: (0, 0)),
        ],
        out_specs=pl.BlockSpec((BLK4, CHUNK), lambda i: (i, 0)),
        out_shape=jax.ShapeDtypeStruct((NP // 4, CHUNK), jnp.float32),
    )(acc4, g4, dinv4, b4, od, W4n)


def _tc_readout_body(acc_ref, g_ref, dinv_ref, b_ref, od_ref, batch_ref,
                     linw_ref, linb_ref, out_ref, mx_s, sm_s, ct_s):
    i = pl.program_id(0)
    nblk = pl.num_programs(0)

    @pl.when(i == 0)
    def _init():
        mx_s[...] = jnp.zeros_like(mx_s)
        sm_s[...] = jnp.zeros_like(sm_s)
        ct_s[...] = jnp.zeros_like(ct_s)

    o = _norm_relu4(acc_ref, g_ref, dinv_ref, b_ref, od_ref)
    batch = batch_ref[...]

    iota = lax.broadcasted_iota(jnp.int32, (BLK4, GG), 1)
    ones32 = jnp.ones((BLK4, HH), jnp.float32)
    dims = (((0,), (0,)), ((), ()))
    sm_c = jnp.zeros((GG, HH), jnp.float32)
    ct_c = jnp.zeros((GG, HH), jnp.float32)
    for k in range(4):
        mk = (batch[:, HH * k:HH * k + 1] == iota).astype(jnp.float32)
        ok = o[:, HH * k:HH * (k + 1)]
        sm_c = sm_c + lax.dot_general(
            mk, ok, dims, preferred_element_type=jnp.float32)
        ct_c = ct_c + lax.dot_general(
            mk, ones32, dims, preferred_element_type=jnp.float32)
    sm_s[...] = sm_s[...] + sm_c
    ct_s[...] = ct_s[...] + ct_c

    lo = batch_ref[0, 0]
    hi = batch_ref[BLK4 - 1, CHUNK - 1]

    def fold(v, op):
        return op(op(v[:, 0:HH], v[:, HH:2 * HH]),
                  op(v[:, 2 * HH:3 * HH], v[:, 3 * HH:4 * HH]))

    def body(gid, carry):
        vals = jnp.where(batch == gid, o, 0.0)
        mx = fold(jnp.max(vals, axis=0, keepdims=True), jnp.maximum)
        mx_s[pl.ds(gid, 1), :] = jnp.maximum(mx_s[pl.ds(gid, 1), :], mx)
        return carry

    lax.fori_loop(lo, hi + 1, body, 0)

    @pl.when(i == nblk - 1)
    def _fin():
        mean = sm_s[...] / jnp.maximum(ct_s[...], 1.0)
        pooled = jnp.concatenate([mx_s[0:GG, :], mean], axis=1)
        out_ref[...] = jnp.dot(
            pooled, linw_ref[...], preferred_element_type=jnp.float32
        ) + linb_ref[...]


def _tc_readout(acc4, g4, dinv4, b4, od, batch4, linW, linb):
    return pl.pallas_call(
        _tc_readout_body,
        grid=(NP // 4 // BLK4,),
        in_specs=[
            pl.BlockSpec((NC, BLK4, CHUNK), lambda i: (0, i, 0)),
            pl.BlockSpec((BLK4, CHUNK), lambda i: (i, 0)),
            pl.BlockSpec((BLK4, CHUNK), lambda i: (i, 0)),
            pl.BlockSpec((1, CHUNK), lambda i: (0, 0)),
            pl.BlockSpec((CHUNK, CHUNK), lambda i: (0, 0)),
            pl.BlockSpec((BLK4, CHUNK), lambda i: (i, 0)),
            pl.BlockSpec((2 * HH, CC), lambda i: (0, 0)),
            pl.BlockSpec((1, CC), lambda i: (0, 0)),
        ],
        out_specs=pl.BlockSpec((GG, CC), lambda i: (0, 0)),
        out_shape=jax.ShapeDtypeStruct((GG, CC), jnp.float32),
        scratch_shapes=[
            pltpu.VMEM((GG + 8, HH), jnp.float32),
            pltpu.VMEM((GG, HH), jnp.float32),
            pltpu.VMEM((GG, HH), jnp.float32),
        ],
    )(acc4, g4, dinv4, b4, od, batch4, linW, linb)


def kernel(x, edge_index, batch, W1, b1, W2, b2, W3, b3, linW, linb):
    f32 = jnp.float32
    xp4 = jnp.zeros((NP, DI), f32).at[:NN].set(x).reshape(NP // 4, 4 * DI)
    batch4 = jnp.broadcast_to(
        jnp.full((NP,), GG, jnp.int32).at[:NN].set(batch)[:, None],
        (NP, HH)).reshape(NP // 4, CHUNK)
    zeros_nh = jnp.zeros((NP, HH), f32)
    zeros_n = jnp.zeros((NP,), f32)
    ei2d = edge_index.reshape(2 * ECH, CHUNK)

    eye4 = jnp.eye(4, dtype=f32)
    od = jnp.kron(eye4, jnp.ones((HH, HH), f32))
    W4_1 = jnp.kron(eye4, W1)
    W4_2 = jnp.kron(eye4, W2)
    W4_3 = jnp.kron(eye4, W3)
    S1 = jnp.kron(jnp.eye(BLK4, dtype=f32), jnp.ones((1, 4), f32))
    S2 = jnp.kron(jnp.ones((BLK4, 1), f32),
                  jnp.kron(eye4, jnp.ones((1, HH), f32)))

    def b4(b):
        return jnp.tile(b, 4).reshape(1, CHUNK)

    def wide(a):
        return a.reshape(a.shape[:-2] + (NP // 4, CHUNK))

    deg2 = _sc_deg(ei2d, zeros_n)
    g1w, dinv4 = _tc_pre(deg2, xp4, W4_1, S1, S2)
    acc1 = _sc_edge(ei2d, g1w.reshape(NP, HH), zeros_nh)
    g2w = _tc_post(wide(acc1), g1w, dinv4, b4(b1), od, W4_2)
    acc2 = _sc_edge(ei2d, g2w.reshape(NP, HH), zeros_nh)
    g3w = _tc_post(wide(acc2), g2w, dinv4, b4(b2), od, W4_3)
    acc3 = _sc_edge(ei2d, g3w.reshape(NP, HH), zeros_nh)
    return _tc_readout(wide(acc3), g3w, dinv4, b4(b3), od, batch4,
                       linW, linb.reshape(1, CC))


# speedup vs baseline: 1.1223x; 1.0148x over previous
import functools
import jax
import jax.numpy as jnp
from jax import lax
from jax.experimental import pallas as pl
from jax.experimental.pallas import tpu as pltpu
from jax.experimental.pallas import tpu_sc as plsc

NN = 10000
NP = 10240
EE = 320000
DI = 128
HH = 32
CC = 10
GG = 64

NC = 2
NS = 16
NW = NC * NS
CHUNK = 128
ECH = EE // CHUNK
CPW = 80
CPW_LAST = ECH - (NW - 1) * CPW
RPS = NP // NS
NBUF = 10

_SC_MESH = plsc.VectorSubcoreMesh(core_axis_name="c", subcore_axis_name="s")
_SC_PARAMS = pltpu.CompilerParams(use_tc_tiling_on_sc=False)


def _worker_info(c, s):
    wid = s * NC + c
    trips = jnp.where(wid == NW - 1, CPW_LAST, CPW)
    return wid, trips


def _sc_deg_body(ei_hbm, zeros_hbm, deg_hbm, idx_dst, ones_v, deg_sp, sem):
    c = lax.axis_index("c")
    s = lax.axis_index("s")
    wid, trips = _worker_info(c, s)
    pltpu.sync_copy(zeros_hbm.at[pl.ds(s * RPS, RPS)],
                    deg_sp.at[pl.ds(s * RPS, RPS)])

    @pl.when(wid < NW - 1)
    def _full():
        pltpu.async_copy(ei_hbm.at[pl.ds(ECH + wid * CPW, CPW)], idx_dst,
                         sem).wait()

    @pl.when(wid == NW - 1)
    def _tail():
        pltpu.async_copy(ei_hbm.at[pl.ds(ECH + wid * CPW, CPW_LAST)],
                         idx_dst.at[pl.ds(0, CPW_LAST)], sem).wait()

    for i in range(CHUNK // 16):
        ones_v[pl.ds(i * 16, 16)] = jnp.ones((16,), jnp.float32)
    plsc.subcore_barrier()

    def body(j, carry):
        pltpu.async_copy(ones_v, deg_sp.at[idx_dst.at[j]], sem, add=True)
        return carry

    lax.fori_loop(0, trips, body, 0)

    def drain(j, carry):
        pltpu.make_async_copy(ones_v, deg_sp.at[idx_dst.at[0]], sem).wait()
        return carry

    lax.fori_loop(0, trips, drain, 0)
    plsc.subcore_barrier()
    pltpu.sync_copy(deg_sp.at[pl.ds(s * RPS, RPS)],
                    deg_hbm.at[c, pl.ds(s * RPS, RPS)])


_sc_deg = functools.partial(
    pl.kernel,
    out_type=jax.ShapeDtypeStruct((NC, NP), jnp.float32),
    mesh=_SC_MESH,
    scratch_types=[
        pltpu.VMEM((CPW, CHUNK), jnp.int32),
        pltpu.VMEM((CHUNK,), jnp.float32),
        pltpu.VMEM_SHARED((NP,), jnp.float32),
        pltpu.SemaphoreType.DMA,
    ],
    compiler_params=_SC_PARAMS,
)(_sc_deg_body)


def _sc_edge_body(ei_hbm, g_hbm, zeros_hbm, acc_hbm,
                  idx_src, idx_dst, rows, acc_sp, gsems, ssems):
    c = lax.axis_index("c")
    s = lax.axis_index("s")
    wid, trips = _worker_info(c, s)
    g_nodes = g_hbm
    pltpu.sync_copy(zeros_hbm.at[pl.ds(s * RPS, RPS)],
                    acc_sp.at[pl.ds(s * RPS, RPS)])

    @pl.when(wid < NW - 1)
    def _full():
        pltpu.async_copy(ei_hbm.at[pl.ds(wid * CPW, CPW)], idx_src,
                         gsems.at[0]).wait()
        pltpu.async_copy(ei_hbm.at[pl.ds(ECH + wid * CPW, CPW)], idx_dst,
                         gsems.at[0]).wait()

    @pl.when(wid == NW - 1)
    def _tail():
        pltpu.async_copy(ei_hbm.at[pl.ds(ECH + wid * CPW, CPW_LAST)],
                         idx_dst.at[pl.ds(0, CPW_LAST)], gsems.at[0]).wait()
        pltpu.async_copy(ei_hbm.at[pl.ds(wid * CPW, CPW_LAST)],
                         idx_src.at[pl.ds(0, CPW_LAST)], gsems.at[0]).wait()

    plsc.subcore_barrier()
    for b in range(NBUF):
        pltpu.async_copy(g_nodes.at[idx_src.at[b]], rows.at[b], gsems.at[b])

    def body(i, carry):
        for b in range(NBUF):
            j = i * NBUF + b
            pltpu.make_async_copy(g_nodes.at[idx_src.at[0]], rows.at[b],
                                  gsems.at[b]).wait()
            pltpu.async_copy(rows.at[b], acc_sp.at[idx_dst.at[j]],
                             ssems.at[b], add=True)
        for b in range(NBUF):
            j = i * NBUF + b
            pltpu.make_async_copy(rows.at[b], acc_sp.at[pl.ds(0, CHUNK)],
                                  ssems.at[b]).wait()

            @pl.when(j + NBUF < trips)
            def _refill():
                pltpu.async_copy(g_nodes.at[idx_src.at[j + NBUF]], rows.at[b],
                                 gsems.at[b])
        return carry

    lax.fori_loop(0, trips // NBUF, body, 0)
    plsc.subcore_barrier()
    pltpu.sync_copy(acc_sp.at[pl.ds(s * RPS, RPS)],
                    acc_hbm.at[c, pl.ds(s * RPS, RPS)])


_sc_edge = functools.partial(
    pl.kernel,
    out_type=jax.ShapeDtypeStruct((NC, NP, HH), jnp.float32),
    mesh=_SC_MESH,
    scratch_types=[
        pltpu.VMEM((CPW, CHUNK), jnp.int32),
        pltpu.VMEM((CPW, CHUNK), jnp.int32),
        pltpu.VMEM((NBUF, CHUNK, HH), jnp.float32),
        pltpu.VMEM_SHARED((NP, HH), jnp.float32),
        pltpu.SemaphoreType.DMA((NBUF,)),
        pltpu.SemaphoreType.DMA((NBUF,)),
    ],
    compiler_params=_SC_PARAMS,
)(_sc_edge_body)


BLK4 = 256


def _tc_pre_body(deg_ref, x_ref, w_ref, s1_ref, s2_ref, g_ref, dinv_ref):
    dinv = lax.rsqrt(deg_ref[0] + deg_ref[1] + 1.0).reshape(4 * BLK4, 1)
    dinv4 = jnp.dot(s1_ref[...], dinv * s2_ref[...],
                    preferred_element_type=jnp.float32)
    h = jnp.dot(x_ref[...], w_ref[...], preferred_element_type=jnp.float32)
    dinv_ref[...] = dinv4
    g_ref[...] = dinv4 * h


def _tc_pre(deg2, xp4, W4, S1, S2):
    return pl.pallas_call(
        _tc_pre_body,
        grid=(NP // 4 // BLK4,),
        in_specs=[
            pl.BlockSpec((NC, 4 * BLK4), lambda i: (0, i)),
            pl.BlockSpec((BLK4, 4 * DI), lambda i: (i, 0)),
            pl.BlockSpec((4 * DI, CHUNK), lambda i: (0, 0)),
            pl.BlockSpec((BLK4, 4 * BLK4), lambda i: (0, 0)),
            pl.BlockSpec((4 * BLK4, CHUNK), lambda i: (0, 0)),
        ],
        out_specs=[
            pl.BlockSpec((BLK4, CHUNK), lambda i: (i, 0)),
            pl.BlockSpec((BLK4, CHUNK), lambda i: (i, 0)),
        ],
        out_shape=[
            jax.ShapeDtypeStruct((NP // 4, CHUNK), jnp.float32),
            jax.ShapeDtypeStruct((NP // 4, CHUNK), jnp.float32),
        ],
    )(deg2, xp4, W4, S1, S2)


def _norm_relu4(acc_ref, g_ref, dinv_ref, b_ref, od_ref):
    g = g_ref[...]
    pre = dinv_ref[...] * (acc_ref[0] + acc_ref[1] + g) + b_ref[...]
    nrm2 = jnp.dot(pre * pre, od_ref[...], preferred_element_type=jnp.float32)
    nrm = jnp.sqrt(nrm2)
    return jnp.maximum(pre / jnp.maximum(nrm, 1e-12), 0.0)


def _tc_post_body(acc_ref, g_ref, dinv_ref, b_ref, od_ref, w_ref, gnext_ref):
    o = _norm_relu4(acc_ref, g_ref, dinv_ref, b_ref, od_ref)
    gnext_ref[...] = dinv_ref[...] * jnp.dot(
        o, w_ref[...], preferred_element_type=jnp.float32)


def _tc_post(acc4, g4, dinv4, b4, od, W4n):
    return pl.pallas_call(
        _tc_post_body,
        grid=(NP // 4 // BLK4,),
        in_specs=[
            pl.BlockSpec((NC, BLK4, CHUNK), lambda i: (0, i, 0)),
            pl.BlockSpec((BLK4, CHUNK), lambda i: (i, 0)),
            pl.BlockSpec((BLK4, CHUNK), lambda i: (i, 0)),
            pl.BlockSpec((1, CHUNK), lambda i: (0, 0)),
            pl.BlockSpec((CHUNK, CHUNK), lambda i: (0, 0)),
            pl.BlockSpec((CHUNK, CHUNK), lambda i: (0, 0)),
        ],
        out_specs=pl.BlockSpec((BLK4, CHUNK), lambda i: (i, 0)),
        out_shape=jax.ShapeDtypeStruct((NP // 4, CHUNK), jnp.float32),
    )(acc4, g4, dinv4, b4, od, W4n)


def _tc_readout_body(acc_ref, g_ref, dinv_ref, b_ref, od_ref, batch_ref,
                     linw_ref, linb_ref, out_ref, mx_s, sm_s, ct_s):
    i = pl.program_id(0)
    nblk = pl.num_programs(0)

    @pl.when(i == 0)
    def _init():
        mx_s[...] = jnp.zeros_like(mx_s)
        sm_s[...] = jnp.zeros_like(sm_s)
        ct_s[...] = jnp.zeros_like(ct_s)

    o = _norm_relu4(acc_ref, g_ref, dinv_ref, b_ref, od_ref)
    batch = batch_ref[...]

    iota = lax.broadcasted_iota(jnp.int32, (BLK4, GG), 1)
    ones32 = jnp.ones((BLK4, HH), jnp.float32)
    dims = (((0,), (0,)), ((), ()))
    sm_c = jnp.zeros((GG, HH), jnp.float32)
    ct_c = jnp.zeros((GG, HH), jnp.float32)
    for k in range(4):
        mk = (batch[:, HH * k:HH * k + 1] == iota).astype(jnp.float32)
        ok = o[:, HH * k:HH * (k + 1)]
        sm_c = sm_c + lax.dot_general(
            mk, ok, dims, preferred_element_type=jnp.float32)
        ct_c = ct_c + lax.dot_general(
            mk, ones32, dims, preferred_element_type=jnp.float32)
    sm_s[...] = sm_s[...] + sm_c
    ct_s[...] = ct_s[...] + ct_c

    lo = batch_ref[0, 0]
    hi = batch_ref[BLK4 - 1, CHUNK - 1]

    def fold(v, op):
        return op(op(v[:, 0:HH], v[:, HH:2 * HH]),
                  op(v[:, 2 * HH:3 * HH], v[:, 3 * HH:4 * HH]))

    def body(gid, carry):
        vals = jnp.where(batch == gid, o, 0.0)
        mx = fold(jnp.max(vals, axis=0, keepdims=True), jnp.maximum)
        mx_s[pl.ds(gid, 1), :] = jnp.maximum(mx_s[pl.ds(gid, 1), :], mx)
        return carry

    lax.fori_loop(lo, hi + 1, body, 0)

    @pl.when(i == nblk - 1)
    def _fin():
        mean = sm_s[...] / jnp.maximum(ct_s[...], 1.0)
        pooled = jnp.concatenate([mx_s[0:GG, :], mean], axis=1)
        out_ref[...] = jnp.dot(
            pooled, linw_ref[...], preferred_element_type=jnp.float32
        ) + linb_ref[...]


def _tc_readout(acc4, g4, dinv4, b4, od, batch4, linW, linb):
    return pl.pallas_call(
        _tc_readout_body,
        grid=(NP // 4 // BLK4,),
        in_specs=[
            pl.BlockSpec((NC, BLK4, CHUNK), lambda i: (0, i, 0)),
            pl.BlockSpec((BLK4, CHUNK), lambda i: (i, 0)),
            pl.BlockSpec((BLK4, CHUNK), lambda i: (i, 0)),
            pl.BlockSpec((1, CHUNK), lambda i: (0, 0)),
            pl.BlockSpec((CHUNK, CHUNK), lambda i: (0, 0)),
            pl.BlockSpec((BLK4, CHUNK), lambda i: (i, 0)),
            pl.BlockSpec((2 * HH, CC), lambda i: (0, 0)),
            pl.BlockSpec((1, CC), lambda i: (0, 0)),
        ],
        out_specs=pl.BlockSpec((GG, CC), lambda i: (0, 0)),
        out_shape=jax.ShapeDtypeStruct((GG, CC), jnp.float32),
        scratch_shapes=[
            pltpu.VMEM((GG + 8, HH), jnp.float32),
            pltpu.VMEM((GG, HH), jnp.float32),
            pltpu.VMEM((GG, HH), jnp.float32),
        ],
    )(acc4, g4, dinv4, b4, od, batch4, linW, linb)


def kernel(x, edge_index, batch, W1, b1, W2, b2, W3, b3, linW, linb):
    f32 = jnp.float32
    xp4 = jnp.zeros((NP, DI), f32).at[:NN].set(x).reshape(NP // 4, 4 * DI)
    batch4 = jnp.broadcast_to(
        jnp.full((NP,), GG, jnp.int32).at[:NN].set(batch)[:, None],
        (NP, HH)).reshape(NP // 4, CHUNK)
    zeros_nh = jnp.zeros((NP, HH), f32)
    zeros_n = jnp.zeros((NP,), f32)
    ei2d = edge_index.reshape(2 * ECH, CHUNK)

    eye4 = jnp.eye(4, dtype=f32)
    od = jnp.kron(eye4, jnp.ones((HH, HH), f32))
    W4_1 = jnp.kron(eye4, W1)
    W4_2 = jnp.kron(eye4, W2)
    W4_3 = jnp.kron(eye4, W3)
    S1 = jnp.kron(jnp.eye(BLK4, dtype=f32), jnp.ones((1, 4), f32))
    S2 = jnp.kron(jnp.ones((BLK4, 1), f32),
                  jnp.kron(eye4, jnp.ones((1, HH), f32)))

    def b4(b):
        return jnp.tile(b, 4).reshape(1, CHUNK)

    def wide(a):
        return a.reshape(a.shape[:-2] + (NP // 4, CHUNK))

    deg2 = _sc_deg(ei2d, zeros_n)
    g1w, dinv4 = _tc_pre(deg2, xp4, W4_1, S1, S2)
    acc1 = _sc_edge(ei2d, g1w.reshape(NP, HH), zeros_nh)
    g2w = _tc_post(wide(acc1), g1w, dinv4, b4(b1), od, W4_2)
    acc2 = _sc_edge(ei2d, g2w.reshape(NP, HH), zeros_nh)
    g3w = _tc_post(wide(acc2), g2w, dinv4, b4(b2), od, W4_3)
    acc3 = _sc_edge(ei2d, g3w.reshape(NP, HH), zeros_nh)
    return _tc_readout(wide(acc3), g3w, dinv4, b4(b3), od, batch4,
                       linW, linb.reshape(1, CC))
